# Initial kernel scaffold; baseline (speedup 1.0000x reference)
import jax
import jax.numpy as jnp
from jax.experimental import pallas as pl


def kernel(field):
    raise NotImplementedError("write your pallas kernel here")


# speedup vs baseline: 1162.5346x; 1162.5346x over previous
import functools

import jax
import jax.numpy as jnp
import numpy as np
from jax.experimental import pallas as pl
from jax.experimental.pallas import tpu as pltpu

_B = 2
_N = 96
_F = 16
_C = 3
_BX = 8
_ZC = _N * _C


def _interp_matrix():
    x = np.linspace(1.0, float(_F) - 2.0, _N).astype(np.float32)
    f = np.floor(x)
    i0 = np.clip(f.astype(np.int64), 0, _F - 1)
    i1 = np.clip(f.astype(np.int64) + 1, 0, _F - 1)
    w = (x - f).astype(np.float32)
    a = np.zeros((_N, _F), dtype=np.float32)
    np.add.at(a, (np.arange(_N), i0), 1.0 - w)
    np.add.at(a, (np.arange(_N), i1), w)
    return a


_A = _interp_matrix()
_WZC = (_A.T[:, None, :, None] * np.eye(_C, dtype=np.float32)[None, :, None, :]
        ).reshape(_F * _C, _ZC)


_PREC = jax.lax.Precision.HIGHEST


def _warp_kernel(f_ref, wzc_ref, ay_ref, ax_ref, o_ref, t_ref):
    @pl.when(pl.program_id(0) == 0)
    def _build_scratch():
        for b in range(_B):
            for i in range(_F):
                v = jnp.dot(f_ref[b, i], wzc_ref[...], precision=_PREC,
                            preferred_element_type=jnp.float32)
                t_ref[b, i] = jnp.dot(ay_ref[...], v, precision=_PREC,
                                      preferred_element_type=jnp.float32)

    axb = ax_ref[...]
    for b in range(_B):
        o_ref[b] = jax.lax.dot_general(
            axb, t_ref[b], (((1,), (0,)), ((), ())), precision=_PREC,
            preferred_element_type=jnp.float32)


@jax.jit
def kernel(field):
    f_r = field.reshape(_B, _F, _F, _F * _C)
    out = pl.pallas_call(
        _warp_kernel,
        grid=(_N // _BX,),
        in_specs=[
            pl.BlockSpec((_B, _F, _F, _F * _C), lambda g: (0, 0, 0, 0)),
            pl.BlockSpec((_F * _C, _ZC), lambda g: (0, 0)),
            pl.BlockSpec((_N, _F), lambda g: (0, 0)),
            pl.BlockSpec((_BX, _F), lambda g: (g, 0)),
        ],
        out_specs=pl.BlockSpec((_B, _BX, _N, _ZC), lambda g: (0, g, 0, 0)),
        out_shape=jax.ShapeDtypeStruct((_B, _N, _N, _ZC), jnp.float32),
        scratch_shapes=[pltpu.VMEM((_B, _F, _N, _ZC), jnp.float32)],
    )(f_r, jnp.asarray(_WZC), jnp.asarray(_A), jnp.asarray(_A))
    return out.reshape(_B, _N, _N, _N, _C)


# speedup vs baseline: 3222.3148x; 2.7718x over previous
import jax
import jax.numpy as jnp
import numpy as np
from jax.experimental import pallas as pl
from jax.experimental.pallas import tpu as pltpu

_B = 2
_N = 96
_F = 16
_C = 3
_BX = 8


def _interp_matrix():
    x = np.linspace(1.0, float(_F) - 2.0, _N).astype(np.float32)
    f = np.floor(x)
    i0 = np.clip(f.astype(np.int64), 0, _F - 1)
    i1 = np.clip(f.astype(np.int64) + 1, 0, _F - 1)
    w = (x - f).astype(np.float32)
    a = np.zeros((_N, _F), dtype=np.float32)
    np.add.at(a, (np.arange(_N), i0), 1.0 - w)
    np.add.at(a, (np.arange(_N), i1), w)
    return a


_A = _interp_matrix()
_PREC = jax.lax.Precision.HIGHEST


def _warp_kernel(f_ref, azt_ref, ay_ref, ax_ref, o_ref, t_ref):
    @pl.when(pl.program_id(0) == 0)
    def _build_scratch():
        for b in range(_B):
            for c in range(_C):
                fbc = f_ref[b, :, c].reshape(_F * _F, _F)
                v = jnp.dot(fbc, azt_ref[...], precision=_PREC,
                            preferred_element_type=jnp.float32)
                for i in range(_F):
                    t_ref[b, c, i] = jnp.dot(
                        ay_ref[...], v[i * _F:(i + 1) * _F], precision=_PREC,
                        preferred_element_type=jnp.float32)

    axb = ax_ref[...]
    for b in range(_B):
        for c in range(_C):
            o_ref[b, :, c] = jax.lax.dot_general(
                axb, t_ref[b, c], (((1,), (0,)), ((), ())), precision=_PREC,
                preferred_element_type=jnp.float32)


@jax.jit
def kernel(field):
    ft = jnp.transpose(field, (0, 1, 4, 2, 3))
    out = pl.pallas_call(
        _warp_kernel,
        grid=(_N // _BX,),
        in_specs=[
            pl.BlockSpec((_B, _F, _C, _F, _F), lambda g: (0, 0, 0, 0, 0)),
            pl.BlockSpec((_F, _N), lambda g: (0, 0)),
            pl.BlockSpec((_N, _F), lambda g: (0, 0)),
            pl.BlockSpec((_BX, _F), lambda g: (g, 0)),
        ],
        out_specs=pl.BlockSpec((_B, _BX, _C, _N, _N),
                               lambda g: (0, g, 0, 0, 0)),
        out_shape=jax.ShapeDtypeStruct((_B, _N, _C, _N, _N), jnp.float32),
        scratch_shapes=[pltpu.VMEM((_B, _C, _F, _N, _N), jnp.float32)],
    )(ft, jnp.asarray(_A.T), jnp.asarray(_A), jnp.asarray(_A))
    return jnp.transpose(out, (0, 1, 3, 4, 2))


# speedup vs baseline: 6723.6216x; 2.0866x over previous
import jax
import jax.numpy as jnp
import numpy as np
from jax.experimental import pallas as pl
from jax.experimental.pallas import tpu as pltpu

_B = 2
_N = 96
_F = 16
_C = 3


def _interp_matrix():
    x = np.linspace(1.0, float(_F) - 2.0, _N).astype(np.float32)
    f = np.floor(x)
    i0 = np.clip(f.astype(np.int64), 0, _F - 1)
    i1 = np.clip(f.astype(np.int64) + 1, 0, _F - 1)
    w = (x - f).astype(np.float32)
    a = np.zeros((_N, _F), dtype=np.float32)
    np.add.at(a, (np.arange(_N), i0), 1.0 - w)
    np.add.at(a, (np.arange(_N), i1), w)
    return a


_A = _interp_matrix()
_PREC = jax.lax.Precision.HIGHEST


def _warp_kernel(f_ref, azt_ref, ay_ref, ax_ref, o_ref, t_ref):
    fbc = f_ref[0, :, 0].reshape(_F * _F, _F)
    v = jnp.dot(fbc, azt_ref[...], precision=_PREC,
                preferred_element_type=jnp.float32)
    for i in range(_F):
        t_ref[i] = jnp.dot(ay_ref[...], v[i * _F:(i + 1) * _F],
                           precision=_PREC,
                           preferred_element_type=jnp.float32)
    o_ref[0, :, 0] = jax.lax.dot_general(
        ax_ref[...], t_ref[...], (((1,), (0,)), ((), ())), precision=_PREC,
        preferred_element_type=jnp.float32)


@jax.jit
def kernel(field):
    ft = jnp.transpose(field, (0, 1, 4, 2, 3))
    out = pl.pallas_call(
        _warp_kernel,
        grid=(_B, _C),
        in_specs=[
            pl.BlockSpec((1, _F, 1, _F, _F), lambda b, c: (b, 0, c, 0, 0)),
            pl.BlockSpec((_F, _N), lambda b, c: (0, 0)),
            pl.BlockSpec((_N, _F), lambda b, c: (0, 0)),
            pl.BlockSpec((_N, _F), lambda b, c: (0, 0)),
        ],
        out_specs=pl.BlockSpec((1, _N, 1, _N, _N),
                               lambda b, c: (b, 0, c, 0, 0)),
        out_shape=jax.ShapeDtypeStruct((_B, _N, _C, _N, _N), jnp.float32),
        scratch_shapes=[pltpu.VMEM((_F, _N, _N), jnp.float32)],
    )(ft, jnp.asarray(_A.T), jnp.asarray(_A), jnp.asarray(_A))
    return jnp.transpose(out, (0, 1, 3, 4, 2))


# speedup vs baseline: 14174.1782x; 2.1081x over previous
import jax
import jax.numpy as jnp
import numpy as np
from jax.experimental import pallas as pl
from jax.experimental.pallas import tpu as pltpu

_B = 2
_N = 96
_F = 16
_C = 3


def _interp_matrix():
    x = np.linspace(1.0, float(_F) - 2.0, _N).astype(np.float32)
    f = np.floor(x)
    i0 = np.clip(f.astype(np.int64), 0, _F - 1)
    i1 = np.clip(f.astype(np.int64) + 1, 0, _F - 1)
    w = (x - f).astype(np.float32)
    a = np.zeros((_N, _F), dtype=np.float32)
    np.add.at(a, (np.arange(_N), i0), 1.0 - w)
    np.add.at(a, (np.arange(_N), i1), w)
    return a


_A = _interp_matrix()
_PREC = jax.lax.Precision.HIGHEST


def _taps():
    x = np.linspace(1.0, float(_F) - 2.0, _N).astype(np.float32)
    f = np.floor(x)
    i0 = np.clip(f.astype(np.int64), 0, _F - 1)
    i1 = np.clip(f.astype(np.int64) + 1, 0, _F - 1)
    w = (x - f).astype(np.float32)
    return [(int(a), int(b), float(ww)) for a, b, ww in zip(i0, i1, w)]


_TAPS = _taps()


def _warp_kernel(f_ref, azt_ref, ay_ref, o_ref, t_ref):
    fbc = f_ref[0, :, 0].reshape(_F * _F, _F)
    v = jnp.dot(fbc, azt_ref[...], precision=_PREC,
                preferred_element_type=jnp.float32)
    for i in range(_F):
        t_ref[i] = jnp.dot(ay_ref[...], v[i * _F:(i + 1) * _F],
                           precision=_PREC,
                           preferred_element_type=jnp.float32)
    for x, (n0, n1, w) in enumerate(_TAPS):
        o_ref[0, x, 0] = (1.0 - w) * t_ref[n0] + w * t_ref[n1]


@jax.jit
def kernel(field):
    ft = jnp.transpose(field, (0, 1, 4, 2, 3))
    out = pl.pallas_call(
        _warp_kernel,
        grid=(_B, _C),
        in_specs=[
            pl.BlockSpec((1, _F, 1, _F, _F), lambda b, c: (b, 0, c, 0, 0)),
            pl.BlockSpec((_F, _N), lambda b, c: (0, 0)),
            pl.BlockSpec((_N, _F), lambda b, c: (0, 0)),
        ],
        out_specs=pl.BlockSpec((1, _N, 1, _N, _N),
                               lambda b, c: (b, 0, c, 0, 0)),
        out_shape=jax.ShapeDtypeStruct((_B, _N, _C, _N, _N), jnp.float32),
        scratch_shapes=[pltpu.VMEM((_F, _N, _N), jnp.float32)],
    )(ft, jnp.asarray(_A.T), jnp.asarray(_A))
    return jnp.transpose(out, (0, 1, 3, 4, 2))
